# baseline (speedup 1.0000x reference)
import dataclasses
import functools

import jax
import jax.numpy as jnp
from jax import lax
from jax.experimental import pallas as pl
from jax.experimental.pallas import tpu as pltpu
from jax.experimental.pallas import tpu_sc as plsc

N = 10000
E = 160000
D = 256
H = 256
HEADS = 4
CLASSES = 40
NEG_SLOPE = 0.2
BN_EPS = 1e-5

NSUB = 16
NROW = N // NSUB
EC = E // 128

_f32 = jnp.float32
_i32 = jnp.int32

_VMESH = plsc.VectorSubcoreMesh(core_axis_name="c", subcore_axis_name="s")

_GDN = lax.GatherDimensionNumbers(
    offset_dims=(), collapsed_slice_dims=(0,), start_index_map=(0,))


def _sc_params():
    cp = pltpu.CompilerParams()
    if "needs_layout_passes" in pltpu.CompilerParams.__dataclass_fields__:
        cp = dataclasses.replace(cp, needs_layout_passes=False)
    return cp


def _lane_bcast(vec, hidx):
    return lax.gather(vec, hidx[:, None], dimension_numbers=_GDN,
                      slice_sizes=(1,),
                      mode=lax.GatherScatterMode.PROMISE_IN_BOUNDS)


def _memset(ref, rows, val):
    @pl.loop(0, rows)
    def _(r):
        @pl.loop(0, 8)
        def _(m):
            ref[r, pl.ds(m * 16, 16)] = jnp.full((16,), val, _f32)


def _shift_idx(dst_ref, src_ref, off):
    for m in range(8):
        dst_ref[0, pl.ds(m * 16, 16)] = src_ref[0, pl.ds(m * 16, 16)] + off


def _own_start(sid):
    return pl.multiple_of(sid * NROW - lax.rem(sid, 8), 8)


def _own_extra(sid):
    return lax.rem(sid, 8) == 7


def _m8(x):
    return pl.multiple_of(x, 8)


def _zero_own(zbuf, ref, start, extra):
    @pl.loop(0, 4)
    def _(i):
        pltpu.sync_copy(zbuf, ref.at[pl.ds(_m8(start + i * 128), 128)])
    pltpu.sync_copy(zbuf.at[pl.ds(0, 112)],
                    ref.at[pl.ds(_m8(start + 512), 112)])

    @pl.when(extra)
    def _():
        pltpu.sync_copy(zbuf.at[pl.ds(0, 8)],
                        ref.at[pl.ds(_m8(start + 624), 8)])


def _copy_own(src_ref, dst_ref, soff, doff, extra):
    pltpu.sync_copy(src_ref.at[pl.ds(_m8(soff), 624)],
                    dst_ref.at[pl.ds(_m8(doff), 624)])

    @pl.when(extra)
    def _():
        pltpu.sync_copy(src_ref.at[pl.ds(_m8(soff + 624), 8)],
                        dst_ref.at[pl.ds(_m8(doff + 624), 8)])


def _sc_sage(table2, src, dst):
    scratch = [
        pltpu.VMEM((1, 128), _i32),
        pltpu.VMEM((1, 128), _i32),
        pltpu.VMEM((1, 128), _i32),
        pltpu.VMEM((1, 128), _i32),
        pltpu.VMEM((128, 128), _f32),
        pltpu.VMEM((128, 128), _f32),
        pltpu.VMEM((1, 128), _i32),
        pltpu.VMEM((1, 128), _i32),
        pltpu.SemaphoreType.DMA,
        pltpu.SemaphoreType.DMA,
        pltpu.SemaphoreType.DMA,
        pltpu.SemaphoreType.DMA,
        pltpu.SemaphoreType.DMA,
        pltpu.SemaphoreType.DMA,
        pltpu.SemaphoreType.DMA,
        pltpu.SemaphoreType.DMA,
        pltpu.VMEM_SHARED((N, 128), _f32),
    ]

    @functools.partial(
        pl.kernel, out_type=jax.ShapeDtypeStruct((2 * N, 128), _f32),
        mesh=_VMESH, scratch_types=scratch, compiler_params=_sc_params(),
    )
    def k(tab_hbm, src_hbm, dst_hbm, out_hbm, sidxA, sidxB, didxA, didxB,
          rowsA, rowsB, sidxPA, sidxPB, gsemA, gsemB, ssemA, ssemB,
          dsemA, dsemB, psemA, psemB, acc):
        cid = lax.axis_index("c")
        sid = lax.axis_index("s")
        start = _own_start(sid)
        extra = _own_extra(sid)

        def prefetch(g, sidxP, psem):
            pltpu.async_copy(src_hbm.at[pl.ds(g * 128, 128)], sidxP.at[0],
                             psem)

        def fire(g, gnext, sidx, sidxP, didx, rows, gsem, dsem, psem):
            base = g * 128
            pltpu.make_async_copy(src_hbm.at[pl.ds(base, 128)],
                                  sidxP.at[0], psem).wait()
            _shift_idx(sidx, sidxP, cid * N)
            pltpu.async_copy(tab_hbm.at[sidx.at[0]], rows, gsem)

            @pl.when(gnext < EC)
            def _():
                prefetch(gnext, sidxP, psem)
            pltpu.async_copy(dst_hbm.at[pl.ds(base, 128)], didx.at[0], dsem)

        def consume(g, sidx, didx, rows, gsem, dsem, ssem):
            base = g * 128
            pltpu.make_async_copy(tab_hbm.at[sidx.at[0]], rows, gsem).wait()
            pltpu.make_async_copy(dst_hbm.at[pl.ds(base, 128)], didx.at[0],
                                  dsem).wait()
            pltpu.async_copy(rows, acc.at[didx.at[0]], ssem, add=True)

        def wait_scatter(didx, rows, ssem):
            pltpu.make_async_copy(rows, acc.at[didx.at[0]], ssem).wait()

        _memset(rowsA, 128, 0.0)
        prefetch(sid, sidxPA, psemA)
        prefetch(sid + 16, sidxPB, psemB)
        _zero_own(rowsA, acc, start, extra)
        plsc.subcore_barrier()

        @pl.loop(0, 40)
        def _(jp):
            gA = sid + 16 * (2 * jp)
            gB = sid + 16 * (2 * jp + 1)

            @pl.when(gA < EC)
            def _():
                @pl.when(jp > 0)
                def _():
                    wait_scatter(didxA, rowsA, ssemA)
                fire(gA, gA + 32, sidxA, sidxPA, didxA, rowsA,
                     gsemA, dsemA, psemA)

            @pl.when(gB < EC)
            def _():
                @pl.when(jp > 0)
                def _():
                    wait_scatter(didxB, rowsB, ssemB)
                fire(gB, gB + 32, sidxB, sidxPB, didxB, rowsB,
                     gsemB, dsemB, psemB)

            @pl.when(gA < EC)
            def _():
                consume(gA, sidxA, didxA, rowsA, gsemA, dsemA, ssemA)

            @pl.when(gB < EC)
            def _():
                consume(gB, sidxB, didxB, rowsB, gsemB, dsemB, ssemB)

        wait_scatter(didxA, rowsA, ssemA)
        wait_scatter(didxB, rowsB, ssemB)

        plsc.subcore_barrier()
        _copy_own(acc, out_hbm, start, cid * N + start, extra)

    return k(table2, src, dst)


def _sc_deg(dst):
    scratch = [
        pltpu.VMEM((1, 128), _i32),
        pltpu.VMEM((128, 128), _f32),
        pltpu.VMEM_SHARED((N, 128), _f32),
    ]

    @functools.partial(
        pl.kernel, out_type=jax.ShapeDtypeStruct((2 * N, 128), _f32),
        mesh=_VMESH, scratch_types=scratch, compiler_params=_sc_params(),
    )
    def k(dst_hbm, out_hbm, didx, ones, acc):
        cid = lax.axis_index("c")
        sid = lax.axis_index("s")
        start = _own_start(sid)
        extra = _own_extra(sid)

        _memset(ones, 128, 0.0)
        _zero_own(ones, acc, start, extra)
        _memset(ones, 128, 1.0)
        plsc.subcore_barrier()

        half = EC // 2

        @pl.loop(0, 40)
        def _(j):
            g = sid + 16 * j

            @pl.when(g < half)
            def _():
                base = (cid * half + g) * 128
                pltpu.sync_copy(dst_hbm.at[pl.ds(base, 128)], didx.at[0])
                pltpu.sync_copy(ones, acc.at[didx.at[0]], add=True)

        plsc.subcore_barrier()
        _copy_own(acc, out_hbm, start, cid * N + start, extra)

    return k(dst)


def _sc_gat_edge(elr2, src, dst, emax):
    out_types = [
        jax.ShapeDtypeStruct((E // 8, 128), _f32),
        jax.ShapeDtypeStruct((2 * N, 128), _f32),
    ]
    scratch = [
        pltpu.VMEM((1, 128), _i32),
        pltpu.VMEM((1, 128), _i32),
        pltpu.VMEM((128, 128), _f32),
        pltpu.VMEM((128, 128), _f32),
        pltpu.VMEM((16, 128), _f32),
        pltpu.VMEM((1, 16), _f32),
        pltpu.VMEM_SHARED((N, 128), _f32),
    ]

    @functools.partial(
        pl.kernel, out_type=out_types, mesh=_VMESH, scratch_types=scratch,
        compiler_params=_sc_params(),
    )
    def k(elr_hbm, src_hbm, dst_hbm, emax_hbm, tnum_hbm, spart_hbm,
          sidx, didx, arows, brows, tb2, emv, sacc):
        cid = lax.axis_index("c")
        sid = lax.axis_index("s")
        start = _own_start(sid)
        extra = _own_extra(sid)

        _memset(brows, 128, 0.0)
        pltpu.sync_copy(emax_hbm, emv)
        _zero_own(brows, sacc, start, extra)
        plsc.subcore_barrier()

        half = EC // 2

        @pl.loop(0, 40)
        def _(j):
            g = sid + 16 * j

            @pl.when(g < half)
            def _():
                gchunk = cid * half + g
                base = gchunk * 128
                pltpu.sync_copy(src_hbm.at[pl.ds(base, 128)], sidx.at[0])
                pltpu.sync_copy(dst_hbm.at[pl.ds(base, 128)], didx.at[0])
                pltpu.sync_copy(elr_hbm.at[sidx.at[0]], arows)
                _shift_idx(sidx, didx, N)
                pltpu.sync_copy(elr_hbm.at[sidx.at[0]], brows)
                ev = emv[0]

                @pl.loop(0, 16)
                def _(eo):
                    for ei in range(8):
                        e = eo * 8 + ei
                        s_ = arows[e, pl.ds(0, 16)] + brows[e, pl.ds(0, 16)]
                        lk = jnp.where(s_ >= 0.0, s_, NEG_SLOPE * s_)
                        t = jnp.exp(lk - ev)
                        tb2[eo, pl.ds(ei * 16, 16)] = t
                        arows[e, pl.ds(0, 16)] = t

                pltpu.sync_copy(tb2, tnum_hbm.at[pl.ds(_m8(gchunk * 16), 16)])
                pltpu.sync_copy(arows, sacc.at[didx.at[0]], add=True)

        plsc.subcore_barrier()
        _copy_own(sacc, spart_hbm, start, cid * N + start, extra)

    return k(elr2, src, dst, emax)


def _sc_gat_msg(feat8, tnum, src, dst):
    scratch = [
        pltpu.VMEM((1, 128), _i32),
        pltpu.VMEM((1, 128), _i32),
        pltpu.VMEM((1, 128), _i32),
        pltpu.VMEM((1, 128), _i32),
        pltpu.VMEM((128, 128), _f32),
        pltpu.VMEM((128, 128), _f32),
        pltpu.VMEM((16, 128), _f32),
        pltpu.VMEM((16, 128), _f32),
        pltpu.VMEM((1, 128), _i32),
        pltpu.VMEM((1, 128), _i32),
        pltpu.SemaphoreType.DMA,
        pltpu.SemaphoreType.DMA,
        pltpu.SemaphoreType.DMA,
        pltpu.SemaphoreType.DMA,
        pltpu.SemaphoreType.DMA,
        pltpu.SemaphoreType.DMA,
        pltpu.SemaphoreType.DMA,
        pltpu.SemaphoreType.DMA,
        pltpu.SemaphoreType.DMA,
        pltpu.SemaphoreType.DMA,
        pltpu.VMEM_SHARED((N, 128), _f32),
    ]

    @functools.partial(
        pl.kernel, out_type=jax.ShapeDtypeStruct((8 * N, 128), _f32),
        mesh=_VMESH, scratch_types=scratch, compiler_params=_sc_params(),
    )
    def k(feat_hbm, tnum_hbm, src_hbm, dst_hbm, out_hbm,
          sidxA, sidxB, didxA, didxB, frowsA, frowsB, tb2A, tb2B,
          sidxPA, sidxPB, gsemA, gsemB, ssemA, ssemB, dsemA, dsemB,
          tsemA, tsemB, psemA, psemB, acc):
        cid = lax.axis_index("c")
        sid = lax.axis_index("s")
        start = _own_start(sid)
        extra = _own_extra(sid)

        def prefetch(g, sidxP, psem):
            pltpu.async_copy(src_hbm.at[pl.ds(g * 128, 128)], sidxP.at[0],
                             psem)

        for q in range(4):
            kchunk = cid * 4 + q
            head = cid * 2 + (q // 2)
            hidx = jnp.full((16,), head, _i32)

            def fire_loads(g, gnext, sidx, sidxP, didx, frows, tb2,
                           gsem, dsem, tsem, psem):
                base = g * 128
                pltpu.make_async_copy(src_hbm.at[pl.ds(base, 128)],
                                      sidxP.at[0], psem).wait()
                _shift_idx(sidx, sidxP, kchunk * N)
                pltpu.async_copy(feat_hbm.at[sidx.at[0]], frows, gsem)

                @pl.when(gnext < EC)
                def _():
                    prefetch(gnext, sidxP, psem)
                pltpu.async_copy(dst_hbm.at[pl.ds(base, 128)], didx.at[0],
                                 dsem)
                pltpu.async_copy(tnum_hbm.at[pl.ds(_m8(g * 16), 16)], tb2,
                                 tsem)

            def compute_scatter(g, sidx, didx, frows, tb2, gsem, dsem, tsem,
                                ssem):
                base = g * 128
                pltpu.make_async_copy(feat_hbm.at[sidx.at[0]], frows,
                                      gsem).wait()
                pltpu.make_async_copy(tnum_hbm.at[pl.ds(_m8(g * 16), 16)],
                                      tb2, tsem).wait()

                @pl.loop(0, 16)
                def _(eo):
                    for ei in range(8):
                        tbc = _lane_bcast(tb2[eo, pl.ds(ei * 16, 16)], hidx)
                        e = eo * 8 + ei
                        for m in range(8):
                            frows[e, pl.ds(m * 16, 16)] = (
                                tbc * frows[e, pl.ds(m * 16, 16)])

                pltpu.make_async_copy(dst_hbm.at[pl.ds(base, 128)],
                                      didx.at[0], dsem).wait()
                pltpu.async_copy(frows, acc.at[didx.at[0]], ssem, add=True)

            def wait_scatter(didx, frows, ssem):
                pltpu.make_async_copy(frows, acc.at[didx.at[0]],
                                      ssem).wait()

            _memset(frowsA, 128, 0.0)
            prefetch(sid, sidxPA, psemA)
            prefetch(sid + 16, sidxPB, psemB)
            _zero_own(frowsA, acc, start, extra)
            plsc.subcore_barrier()

            @pl.loop(0, 40)
            def _(jp):
                gA = sid + 16 * (2 * jp)
                gB = sid + 16 * (2 * jp + 1)

                @pl.when(gA < EC)
                def _():
                    @pl.when(jp > 0)
                    def _():
                        wait_scatter(didxA, frowsA, ssemA)
                    fire_loads(gA, gA + 32, sidxA, sidxPA, didxA, frowsA,
                               tb2A, gsemA, dsemA, tsemA, psemA)

                @pl.when(gB < EC)
                def _():
                    @pl.when(jp > 0)
                    def _():
                        wait_scatter(didxB, frowsB, ssemB)
                    fire_loads(gB, gB + 32, sidxB, sidxPB, didxB, frowsB,
                               tb2B, gsemB, dsemB, tsemB, psemB)

                @pl.when(gA < EC)
                def _():
                    compute_scatter(gA, sidxA, didxA, frowsA, tb2A,
                                    gsemA, dsemA, tsemA, ssemA)

                @pl.when(gB < EC)
                def _():
                    compute_scatter(gB, sidxB, didxB, frowsB, tb2B,
                                    gsemB, dsemB, tsemB, ssemB)

            wait_scatter(didxA, frowsA, ssemA)
            wait_scatter(didxB, frowsB, ssemB)

            plsc.subcore_barrier()
            _copy_own(acc, out_hbm, start, kchunk * N + start, extra)

    return k(feat8, tnum, src, dst)


_NB = 2000


def _tc_vae(x, W1, b1, Wmu, bmu):
    def body(x_ref, W1_ref, b1_ref, Wmu_ref, bmu_ref, z_ref, x2_ref):
        xb = x_ref[...]
        h = jnp.maximum(
            jnp.dot(xb, W1_ref[...], preferred_element_type=_f32)
            + b1_ref[...], 0.0)
        z_ref[...] = (jnp.dot(h, Wmu_ref[...], preferred_element_type=_f32)
                      + bmu_ref[...])
        x2_ref[0] = xb[:, :128]
        x2_ref[1] = xb[:, 128:]

    return pl.pallas_call(
        body,
        grid=(N // _NB,),
        in_specs=[
            pl.BlockSpec((_NB, D), lambda i: (i, 0)),
            pl.BlockSpec((D, 512), lambda i: (0, 0)),
            pl.BlockSpec((512,), lambda i: (0,)),
            pl.BlockSpec((512, H), lambda i: (0, 0)),
            pl.BlockSpec((H,), lambda i: (0,)),
        ],
        out_specs=[
            pl.BlockSpec((_NB, H), lambda i: (i, 0)),
            pl.BlockSpec((2, _NB, 128), lambda i: (0, i, 0)),
        ],
        out_shape=[
            jax.ShapeDtypeStruct((N, H), _f32),
            jax.ShapeDtypeStruct((2, N, 128), _f32),
        ],
    )(x, W1, b1, Wmu, bmu)


def _tc_sage_post(agg2, xin2, deg2, z, W, b, gamma, beta):
    def mm_body(agg_ref, xin_ref, deg_ref, z_ref, W_ref, b_ref,
                pre_ref, stat_ref):
        i = pl.program_id(0)
        agg = jnp.concatenate([agg_ref[0], agg_ref[1]], axis=1)
        xin = jnp.concatenate([xin_ref[0], xin_ref[1]], axis=1)
        deg = deg_ref[0, :, 0:1] + deg_ref[1, :, 0:1]
        hmean = (agg + xin) / (deg + 1.0)
        pre = (jnp.dot(hmean, W_ref[...], preferred_element_type=_f32)
               + b_ref[...] + z_ref[...])
        pre_ref[0] = pre[:, :128]
        pre_ref[1] = pre[:, 128:]

        @pl.when(i == 0)
        def _():
            stat_ref[...] = jnp.zeros((2, H), _f32)

        cur = jnp.stack([jnp.sum(pre, axis=0), jnp.sum(pre * pre, axis=0)])
        stat_ref[...] = stat_ref[...] + cur

    pre2, stat = pl.pallas_call(
        mm_body,
        grid=(N // _NB,),
        in_specs=[
            pl.BlockSpec((2, _NB, 128), lambda i: (0, i, 0)),
            pl.BlockSpec((2, _NB, 128), lambda i: (0, i, 0)),
            pl.BlockSpec((2, _NB, 128), lambda i: (0, i, 0)),
            pl.BlockSpec((_NB, H), lambda i: (i, 0)),
            pl.BlockSpec((H, H), lambda i: (0, 0)),
            pl.BlockSpec((H,), lambda i: (0,)),
        ],
        out_specs=[
            pl.BlockSpec((2, _NB, 128), lambda i: (0, i, 0)),
            pl.BlockSpec((2, H), lambda i: (0, 0)),
        ],
        out_shape=[
            jax.ShapeDtypeStruct((2, N, 128), _f32),
            jax.ShapeDtypeStruct((2, H), _f32),
        ],
    )(agg2, xin2, deg2, z, W, b)

    def bn_body(pre_ref, stat_ref, g_ref, be_ref, out_ref):
        mu = stat_ref[0:1] / N
        var = stat_ref[1:2] / N - mu * mu
        scale = lax.rsqrt(var + BN_EPS) * g_ref[...]
        shift = be_ref[...] - mu * scale
        pre = jnp.concatenate([pre_ref[0], pre_ref[1]], axis=1)
        hn = jnp.maximum(pre * scale + shift, 0.0)
        out_ref[0] = hn[:, :128]
        out_ref[1] = hn[:, 128:]

    return pl.pallas_call(
        bn_body,
        grid=(N // _NB,),
        in_specs=[
            pl.BlockSpec((2, _NB, 128), lambda i: (0, i, 0)),
            pl.BlockSpec((2, H), lambda i: (0, 0)),
            pl.BlockSpec((H,), lambda i: (0,)),
            pl.BlockSpec((H,), lambda i: (0,)),
        ],
        out_specs=pl.BlockSpec((2, _NB, 128), lambda i: (0, i, 0)),
        out_shape=jax.ShapeDtypeStruct((2, N, 128), _f32),
    )(pre2, stat, gamma, beta)


def _tc_gat_pre(h2, W, al, ar):
    K = h2.shape[0]

    def body(h_ref, W_ref, al_ref, ar_ref, feat_ref, elr_ref, emax_ref,
             mx_ref):
        i = pl.program_id(0)
        hcat = jnp.concatenate([h_ref[j] for j in range(K)], axis=1)
        for kk in range(8):
            feat_ref[kk] = jnp.dot(hcat, W_ref[:, kk * 128:(kk + 1) * 128],
                                   preferred_element_type=_f32)
        Wfull = W_ref[...]
        els, ers = [], []
        for hh in range(HEADS):
            Wh = Wfull[:, hh * H:(hh + 1) * H]
            wl = jnp.dot(Wh, al_ref[hh].reshape(H, 1),
                         preferred_element_type=_f32)
            wr = jnp.dot(Wh, ar_ref[hh].reshape(H, 1),
                         preferred_element_type=_f32)
            els.append(jnp.dot(hcat, wl, preferred_element_type=_f32))
            ers.append(jnp.dot(hcat, wr, preferred_element_type=_f32))
        el = jnp.concatenate(els, axis=1)
        er = jnp.concatenate(ers, axis=1)
        zpad = jnp.zeros((el.shape[0], 124), _f32)
        elr_ref[0] = jnp.concatenate([el, zpad], axis=1)
        elr_ref[1] = jnp.concatenate([er, zpad], axis=1)

        mel = jnp.max(el, axis=0)
        mer = jnp.max(er, axis=0)
        cur = jnp.concatenate([mel, mer]).reshape(1, 8)

        @pl.when(i == 0)
        def _():
            mx_ref[...] = jnp.full((1, 8), -jnp.inf, _f32)

        mx_ref[...] = jnp.maximum(mx_ref[...], cur)

        @pl.when(i == N // _NB - 1)
        def _():
            m = mx_ref[...]
            mh = jnp.maximum(m[:, :4] + m[:, 4:], 0.0)
            emax_ref[...] = jnp.concatenate(
                [mh, jnp.zeros((1, 12), _f32)], axis=1)

    return pl.pallas_call(
        body,
        grid=(N // _NB,),
        in_specs=[
            pl.BlockSpec((K, _NB, 128), lambda i: (0, i, 0)),
            pl.BlockSpec((K * 128, 8 * 128), lambda i: (0, 0)),
            pl.BlockSpec((HEADS, H), lambda i: (0, 0)),
            pl.BlockSpec((HEADS, H), lambda i: (0, 0)),
        ],
        out_specs=[
            pl.BlockSpec((8, _NB, 128), lambda i: (0, i, 0)),
            pl.BlockSpec((2, _NB, 128), lambda i: (0, i, 0)),
            pl.BlockSpec((1, 16), lambda i: (0, 0)),
        ],
        out_shape=[
            jax.ShapeDtypeStruct((8, N, 128), _f32),
            jax.ShapeDtypeStruct((2, N, 128), _f32),
            jax.ShapeDtypeStruct((1, 16), _f32),
        ],
        scratch_shapes=[pltpu.VMEM((1, 8), _f32)],
    )(h2, W, al, ar)


def _tc_gat_post(B8, spart2, bgat):
    def body(B_ref, sp_ref, b_ref, out_ref):
        s = sp_ref[0, :, :4] + sp_ref[1, :, :4]
        rinv = 1.0 / (s + 1e-16)
        for kk in range(8):
            hh = kk // 2
            out_ref[kk] = (B_ref[kk] * rinv[:, hh:hh + 1]
                           + b_ref[0, kk * 128:(kk + 1) * 128])

    return pl.pallas_call(
        body,
        grid=(N // _NB,),
        in_specs=[
            pl.BlockSpec((8, _NB, 128), lambda i: (0, i, 0)),
            pl.BlockSpec((2, _NB, 128), lambda i: (0, i, 0)),
            pl.BlockSpec((1, 8 * 128), lambda i: (0, 0)),
        ],
        out_specs=pl.BlockSpec((8, _NB, 128), lambda i: (0, i, 0)),
        out_shape=jax.ShapeDtypeStruct((8, N, 128), _f32),
    )(B8, spart2, bgat.reshape(1, 8 * 128))


def _tc_classifier(h8, W, b):
    def body(h_ref, W_ref, b_ref, out_ref):
        acc = b_ref[...] + jnp.zeros((h_ref.shape[1], CLASSES), _f32)
        for kk in range(8):
            acc = acc + jnp.dot(h_ref[kk], W_ref[kk * 128:(kk + 1) * 128],
                                preferred_element_type=_f32)
        out_ref[...] = acc

    return pl.pallas_call(
        body,
        grid=(N // _NB,),
        in_specs=[
            pl.BlockSpec((8, _NB, 128), lambda i: (0, i, 0)),
            pl.BlockSpec((8 * 128, CLASSES), lambda i: (0, 0)),
            pl.BlockSpec((CLASSES,), lambda i: (0,)),
        ],
        out_specs=pl.BlockSpec((_NB, CLASSES), lambda i: (i, 0)),
        out_shape=jax.ShapeDtypeStruct((N, CLASSES), _f32),
    )(h8, W, b)


def kernel(x, edge_index, params):
    p = params
    src = edge_index[0].astype(_i32)
    dst = edge_index[1].astype(_i32)

    z, x2 = _tc_vae(x, p['ae_W1'], p['ae_b1'], p['ae_Wmu'], p['ae_bmu'])

    deg2 = _sc_deg(dst).reshape(2, N, 128)

    agg0 = _sc_sage(x2.reshape(2 * N, 128), src, dst)
    h2 = _tc_sage_post(agg0.reshape(2, N, 128), x2, deg2, z,
                       p['sage_W0'], p['sage_b0'], p['bn_g0'], p['bn_b0'])
    agg1 = _sc_sage(h2.reshape(2 * N, 128), src, dst)
    h2 = _tc_sage_post(agg1.reshape(2, N, 128), h2, deg2, z,
                       p['sage_W1'], p['sage_b1'], p['bn_g1'], p['bn_b1'])

    for i in range(2):
        W = p['gat_W%d' % i]
        feat8, elr, emax = _tc_gat_pre(h2, W, p['gat_al%d' % i],
                                       p['gat_ar%d' % i])
        tnum, spart = _sc_gat_edge(elr.reshape(2 * N, 128), src, dst, emax)
        B8 = _sc_gat_msg(feat8.reshape(8 * N, 128), tnum, src, dst)
        h2 = _tc_gat_post(B8.reshape(8, N, 128), spart.reshape(2, N, 128),
                          p['gat_b%d' % i])

    return _tc_classifier(h2, p['cls_W'], p['cls_b'])


# speedup vs baseline: 1.0068x; 1.0068x over previous
import dataclasses
import functools

import jax
import jax.numpy as jnp
from jax import lax
from jax.experimental import pallas as pl
from jax.experimental.pallas import tpu as pltpu
from jax.experimental.pallas import tpu_sc as plsc

N = 10000
E = 160000
D = 256
H = 256
HEADS = 4
CLASSES = 40
NEG_SLOPE = 0.2
BN_EPS = 1e-5

NSUB = 16
NROW = N // NSUB
EC = E // 128

_f32 = jnp.float32
_i32 = jnp.int32

_VMESH = plsc.VectorSubcoreMesh(core_axis_name="c", subcore_axis_name="s")

_GDN = lax.GatherDimensionNumbers(
    offset_dims=(), collapsed_slice_dims=(0,), start_index_map=(0,))


def _sc_params():
    cp = pltpu.CompilerParams()
    if "needs_layout_passes" in pltpu.CompilerParams.__dataclass_fields__:
        cp = dataclasses.replace(cp, needs_layout_passes=False)
    return cp


def _lane_bcast(vec, hidx):
    return lax.gather(vec, hidx[:, None], dimension_numbers=_GDN,
                      slice_sizes=(1,),
                      mode=lax.GatherScatterMode.PROMISE_IN_BOUNDS)


def _memset(ref, rows, val):
    @pl.loop(0, rows)
    def _(r):
        @pl.loop(0, 8)
        def _(m):
            ref[r, pl.ds(m * 16, 16)] = jnp.full((16,), val, _f32)


def _shift_idx(dst_ref, src_ref, off):
    for m in range(8):
        dst_ref[0, pl.ds(m * 16, 16)] = src_ref[0, pl.ds(m * 16, 16)] + off


def _own_start(sid):
    return pl.multiple_of(sid * NROW - lax.rem(sid, 8), 8)


def _own_extra(sid):
    return lax.rem(sid, 8) == 7


def _m8(x):
    return pl.multiple_of(x, 8)


def _zero_own(zbuf, ref, start, extra):
    @pl.loop(0, 4)
    def _(i):
        pltpu.sync_copy(zbuf, ref.at[pl.ds(_m8(start + i * 128), 128)])
    pltpu.sync_copy(zbuf.at[pl.ds(0, 112)],
                    ref.at[pl.ds(_m8(start + 512), 112)])

    @pl.when(extra)
    def _():
        pltpu.sync_copy(zbuf.at[pl.ds(0, 8)],
                        ref.at[pl.ds(_m8(start + 624), 8)])


def _copy_own(src_ref, dst_ref, soff, doff, extra):
    pltpu.sync_copy(src_ref.at[pl.ds(_m8(soff), 624)],
                    dst_ref.at[pl.ds(_m8(doff), 624)])

    @pl.when(extra)
    def _():
        pltpu.sync_copy(src_ref.at[pl.ds(_m8(soff + 624), 8)],
                        dst_ref.at[pl.ds(_m8(doff + 624), 8)])


def _sc_sage(table2, src, dst):
    scratch = [
        pltpu.VMEM((1, 128), _i32),
        pltpu.VMEM((1, 128), _i32),
        pltpu.VMEM((1, 128), _i32),
        pltpu.VMEM((1, 128), _i32),
        pltpu.VMEM((128, 128), _f32),
        pltpu.VMEM((128, 128), _f32),
        pltpu.SemaphoreType.DMA,
        pltpu.SemaphoreType.DMA,
        pltpu.SemaphoreType.DMA,
        pltpu.SemaphoreType.DMA,
        pltpu.SemaphoreType.DMA,
        pltpu.SemaphoreType.DMA,
        pltpu.VMEM_SHARED((N, 128), _f32),
    ]

    @functools.partial(
        pl.kernel, out_type=jax.ShapeDtypeStruct((2 * N, 128), _f32),
        mesh=_VMESH, scratch_types=scratch, compiler_params=_sc_params(),
    )
    def k(tab_hbm, src_hbm, dst_hbm, out_hbm, sidxA, sidxB, didxA, didxB,
          rowsA, rowsB, gsemA, gsemB, ssemA, ssemB, dsemA, dsemB, acc):
        cid = lax.axis_index("c")
        sid = lax.axis_index("s")
        start = _own_start(sid)
        extra = _own_extra(sid)

        def fire(g, sidx, didx, rows, gsem, dsem):
            base = g * 128
            pltpu.sync_copy(src_hbm.at[pl.ds(base, 128)], sidx.at[0])
            _shift_idx(sidx, sidx, cid * N)
            pltpu.async_copy(tab_hbm.at[sidx.at[0]], rows, gsem)
            pltpu.async_copy(dst_hbm.at[pl.ds(base, 128)], didx.at[0], dsem)

        def consume(g, sidx, didx, rows, gsem, dsem, ssem):
            base = g * 128
            pltpu.make_async_copy(tab_hbm.at[sidx.at[0]], rows, gsem).wait()
            pltpu.make_async_copy(dst_hbm.at[pl.ds(base, 128)], didx.at[0],
                                  dsem).wait()
            pltpu.async_copy(rows, acc.at[didx.at[0]], ssem, add=True)

        def wait_scatter(didx, rows, ssem):
            pltpu.make_async_copy(rows, acc.at[didx.at[0]], ssem).wait()

        _memset(rowsA, 128, 0.0)
        _zero_own(rowsA, acc, start, extra)
        plsc.subcore_barrier()

        @pl.loop(0, 40)
        def _(jp):
            gA = sid + 16 * (2 * jp)
            gB = sid + 16 * (2 * jp + 1)

            @pl.when(gA < EC)
            def _():
                @pl.when(jp > 0)
                def _():
                    wait_scatter(didxA, rowsA, ssemA)
                fire(gA, sidxA, didxA, rowsA, gsemA, dsemA)

            @pl.when(gB < EC)
            def _():
                @pl.when(jp > 0)
                def _():
                    wait_scatter(didxB, rowsB, ssemB)
                fire(gB, sidxB, didxB, rowsB, gsemB, dsemB)

            @pl.when(gA < EC)
            def _():
                consume(gA, sidxA, didxA, rowsA, gsemA, dsemA, ssemA)

            @pl.when(gB < EC)
            def _():
                consume(gB, sidxB, didxB, rowsB, gsemB, dsemB, ssemB)

        wait_scatter(didxA, rowsA, ssemA)
        wait_scatter(didxB, rowsB, ssemB)

        plsc.subcore_barrier()
        _copy_own(acc, out_hbm, start, cid * N + start, extra)

    return k(table2, src, dst)


def _sc_deg(dst):
    scratch = [
        pltpu.VMEM((1, 128), _i32),
        pltpu.VMEM((128, 128), _f32),
        pltpu.VMEM_SHARED((N, 128), _f32),
    ]

    @functools.partial(
        pl.kernel, out_type=jax.ShapeDtypeStruct((2 * N, 128), _f32),
        mesh=_VMESH, scratch_types=scratch, compiler_params=_sc_params(),
    )
    def k(dst_hbm, out_hbm, didx, ones, acc):
        cid = lax.axis_index("c")
        sid = lax.axis_index("s")
        start = _own_start(sid)
        extra = _own_extra(sid)

        _memset(ones, 128, 0.0)
        _zero_own(ones, acc, start, extra)
        _memset(ones, 128, 1.0)
        plsc.subcore_barrier()

        half = EC // 2

        @pl.loop(0, 40)
        def _(j):
            g = sid + 16 * j

            @pl.when(g < half)
            def _():
                base = (cid * half + g) * 128
                pltpu.sync_copy(dst_hbm.at[pl.ds(base, 128)], didx.at[0])
                pltpu.sync_copy(ones, acc.at[didx.at[0]], add=True)

        plsc.subcore_barrier()
        _copy_own(acc, out_hbm, start, cid * N + start, extra)

    return k(dst)


def _sc_gat_edge(elr2, src, dst, emax):
    out_types = [
        jax.ShapeDtypeStruct((E // 8, 128), _f32),
        jax.ShapeDtypeStruct((2 * N, 128), _f32),
    ]
    scratch = [
        pltpu.VMEM((1, 128), _i32),
        pltpu.VMEM((1, 128), _i32),
        pltpu.VMEM((128, 128), _f32),
        pltpu.VMEM((128, 128), _f32),
        pltpu.VMEM((16, 128), _f32),
        pltpu.VMEM((1, 16), _f32),
        pltpu.VMEM_SHARED((N, 128), _f32),
    ]

    @functools.partial(
        pl.kernel, out_type=out_types, mesh=_VMESH, scratch_types=scratch,
        compiler_params=_sc_params(),
    )
    def k(elr_hbm, src_hbm, dst_hbm, emax_hbm, tnum_hbm, spart_hbm,
          sidx, didx, arows, brows, tb2, emv, sacc):
        cid = lax.axis_index("c")
        sid = lax.axis_index("s")
        start = _own_start(sid)
        extra = _own_extra(sid)

        _memset(brows, 128, 0.0)
        pltpu.sync_copy(emax_hbm, emv)
        _zero_own(brows, sacc, start, extra)
        plsc.subcore_barrier()

        half = EC // 2

        @pl.loop(0, 40)
        def _(j):
            g = sid + 16 * j

            @pl.when(g < half)
            def _():
                gchunk = cid * half + g
                base = gchunk * 128
                pltpu.sync_copy(src_hbm.at[pl.ds(base, 128)], sidx.at[0])
                pltpu.sync_copy(dst_hbm.at[pl.ds(base, 128)], didx.at[0])
                pltpu.sync_copy(elr_hbm.at[sidx.at[0]], arows)
                _shift_idx(sidx, didx, N)
                pltpu.sync_copy(elr_hbm.at[sidx.at[0]], brows)
                ev = emv[0]

                @pl.loop(0, 16)
                def _(eo):
                    for ei in range(8):
                        e = eo * 8 + ei
                        s_ = arows[e, pl.ds(0, 16)] + brows[e, pl.ds(0, 16)]
                        lk = jnp.where(s_ >= 0.0, s_, NEG_SLOPE * s_)
                        t = jnp.exp(lk - ev)
                        tb2[eo, pl.ds(ei * 16, 16)] = t
                        arows[e, pl.ds(0, 16)] = t

                pltpu.sync_copy(tb2, tnum_hbm.at[pl.ds(_m8(gchunk * 16), 16)])
                pltpu.sync_copy(arows, sacc.at[didx.at[0]], add=True)

        plsc.subcore_barrier()
        _copy_own(sacc, spart_hbm, start, cid * N + start, extra)

    return k(elr2, src, dst, emax)


def _sc_gat_msg(feat8, tnum, src, dst):
    scratch = [
        pltpu.VMEM((1, 128), _i32),
        pltpu.VMEM((1, 128), _i32),
        pltpu.VMEM((1, 128), _i32),
        pltpu.VMEM((1, 128), _i32),
        pltpu.VMEM((128, 128), _f32),
        pltpu.VMEM((128, 128), _f32),
        pltpu.VMEM((16, 128), _f32),
        pltpu.VMEM((16, 128), _f32),
        pltpu.VMEM((1, 128), _i32),
        pltpu.VMEM((1, 128), _i32),
        pltpu.SemaphoreType.DMA,
        pltpu.SemaphoreType.DMA,
        pltpu.SemaphoreType.DMA,
        pltpu.SemaphoreType.DMA,
        pltpu.SemaphoreType.DMA,
        pltpu.SemaphoreType.DMA,
        pltpu.SemaphoreType.DMA,
        pltpu.SemaphoreType.DMA,
        pltpu.SemaphoreType.DMA,
        pltpu.SemaphoreType.DMA,
        pltpu.VMEM_SHARED((N, 128), _f32),
    ]

    @functools.partial(
        pl.kernel, out_type=jax.ShapeDtypeStruct((8 * N, 128), _f32),
        mesh=_VMESH, scratch_types=scratch, compiler_params=_sc_params(),
    )
    def k(feat_hbm, tnum_hbm, src_hbm, dst_hbm, out_hbm,
          sidxA, sidxB, didxA, didxB, frowsA, frowsB, tb2A, tb2B,
          sidxPA, sidxPB, gsemA, gsemB, ssemA, ssemB, dsemA, dsemB,
          tsemA, tsemB, psemA, psemB, acc):
        cid = lax.axis_index("c")
        sid = lax.axis_index("s")
        start = _own_start(sid)
        extra = _own_extra(sid)

        def prefetch(g, sidxP, psem):
            pltpu.async_copy(src_hbm.at[pl.ds(g * 128, 128)], sidxP.at[0],
                             psem)

        for q in range(4):
            kchunk = cid * 4 + q
            head = cid * 2 + (q // 2)
            hidx = jnp.full((16,), head, _i32)

            def fire_loads(g, gnext, sidx, sidxP, didx, frows, tb2,
                           gsem, dsem, tsem, psem):
                base = g * 128
                pltpu.make_async_copy(src_hbm.at[pl.ds(base, 128)],
                                      sidxP.at[0], psem).wait()
                _shift_idx(sidx, sidxP, kchunk * N)
                pltpu.async_copy(feat_hbm.at[sidx.at[0]], frows, gsem)

                @pl.when(gnext < EC)
                def _():
                    prefetch(gnext, sidxP, psem)
                pltpu.async_copy(dst_hbm.at[pl.ds(base, 128)], didx.at[0],
                                 dsem)
                pltpu.async_copy(tnum_hbm.at[pl.ds(_m8(g * 16), 16)], tb2,
                                 tsem)

            def compute_scatter(g, sidx, didx, frows, tb2, gsem, dsem, tsem,
                                ssem):
                base = g * 128
                pltpu.make_async_copy(feat_hbm.at[sidx.at[0]], frows,
                                      gsem).wait()
                pltpu.make_async_copy(tnum_hbm.at[pl.ds(_m8(g * 16), 16)],
                                      tb2, tsem).wait()

                @pl.loop(0, 16)
                def _(eo):
                    for ei in range(8):
                        tbc = _lane_bcast(tb2[eo, pl.ds(ei * 16, 16)], hidx)
                        e = eo * 8 + ei
                        for m in range(8):
                            frows[e, pl.ds(m * 16, 16)] = (
                                tbc * frows[e, pl.ds(m * 16, 16)])

                pltpu.make_async_copy(dst_hbm.at[pl.ds(base, 128)],
                                      didx.at[0], dsem).wait()
                pltpu.async_copy(frows, acc.at[didx.at[0]], ssem, add=True)

            def wait_scatter(didx, frows, ssem):
                pltpu.make_async_copy(frows, acc.at[didx.at[0]],
                                      ssem).wait()

            _memset(frowsA, 128, 0.0)
            prefetch(sid, sidxPA, psemA)
            prefetch(sid + 16, sidxPB, psemB)
            _zero_own(frowsA, acc, start, extra)
            plsc.subcore_barrier()

            @pl.loop(0, 40)
            def _(jp):
                gA = sid + 16 * (2 * jp)
                gB = sid + 16 * (2 * jp + 1)

                @pl.when(gA < EC)
                def _():
                    @pl.when(jp > 0)
                    def _():
                        wait_scatter(didxA, frowsA, ssemA)
                    fire_loads(gA, gA + 32, sidxA, sidxPA, didxA, frowsA,
                               tb2A, gsemA, dsemA, tsemA, psemA)

                @pl.when(gB < EC)
                def _():
                    @pl.when(jp > 0)
                    def _():
                        wait_scatter(didxB, frowsB, ssemB)
                    fire_loads(gB, gB + 32, sidxB, sidxPB, didxB, frowsB,
                               tb2B, gsemB, dsemB, tsemB, psemB)

                @pl.when(gA < EC)
                def _():
                    compute_scatter(gA, sidxA, didxA, frowsA, tb2A,
                                    gsemA, dsemA, tsemA, ssemA)

                @pl.when(gB < EC)
                def _():
                    compute_scatter(gB, sidxB, didxB, frowsB, tb2B,
                                    gsemB, dsemB, tsemB, ssemB)

            wait_scatter(didxA, frowsA, ssemA)
            wait_scatter(didxB, frowsB, ssemB)

            plsc.subcore_barrier()
            _copy_own(acc, out_hbm, start, kchunk * N + start, extra)

    return k(feat8, tnum, src, dst)


_NB = 2000


def _tc_vae(x, W1, b1, Wmu, bmu):
    def body(x_ref, W1_ref, b1_ref, Wmu_ref, bmu_ref, z_ref, x2_ref):
        xb = x_ref[...]
        h = jnp.maximum(
            jnp.dot(xb, W1_ref[...], preferred_element_type=_f32)
            + b1_ref[...], 0.0)
        z_ref[...] = (jnp.dot(h, Wmu_ref[...], preferred_element_type=_f32)
                      + bmu_ref[...])
        x2_ref[0] = xb[:, :128]
        x2_ref[1] = xb[:, 128:]

    return pl.pallas_call(
        body,
        grid=(N // _NB,),
        in_specs=[
            pl.BlockSpec((_NB, D), lambda i: (i, 0)),
            pl.BlockSpec((D, 512), lambda i: (0, 0)),
            pl.BlockSpec((512,), lambda i: (0,)),
            pl.BlockSpec((512, H), lambda i: (0, 0)),
            pl.BlockSpec((H,), lambda i: (0,)),
        ],
        out_specs=[
            pl.BlockSpec((_NB, H), lambda i: (i, 0)),
            pl.BlockSpec((2, _NB, 128), lambda i: (0, i, 0)),
        ],
        out_shape=[
            jax.ShapeDtypeStruct((N, H), _f32),
            jax.ShapeDtypeStruct((2, N, 128), _f32),
        ],
    )(x, W1, b1, Wmu, bmu)


def _tc_sage_post(agg2, xin2, deg2, z, W, b, gamma, beta):
    def mm_body(agg_ref, xin_ref, deg_ref, z_ref, W_ref, b_ref,
                pre_ref, stat_ref):
        i = pl.program_id(0)
        agg = jnp.concatenate([agg_ref[0], agg_ref[1]], axis=1)
        xin = jnp.concatenate([xin_ref[0], xin_ref[1]], axis=1)
        deg = deg_ref[0, :, 0:1] + deg_ref[1, :, 0:1]
        hmean = (agg + xin) / (deg + 1.0)
        pre = (jnp.dot(hmean, W_ref[...], preferred_element_type=_f32)
               + b_ref[...] + z_ref[...])
        pre_ref[0] = pre[:, :128]
        pre_ref[1] = pre[:, 128:]

        @pl.when(i == 0)
        def _():
            stat_ref[...] = jnp.zeros((2, H), _f32)

        cur = jnp.stack([jnp.sum(pre, axis=0), jnp.sum(pre * pre, axis=0)])
        stat_ref[...] = stat_ref[...] + cur

    pre2, stat = pl.pallas_call(
        mm_body,
        grid=(N // _NB,),
        in_specs=[
            pl.BlockSpec((2, _NB, 128), lambda i: (0, i, 0)),
            pl.BlockSpec((2, _NB, 128), lambda i: (0, i, 0)),
            pl.BlockSpec((2, _NB, 128), lambda i: (0, i, 0)),
            pl.BlockSpec((_NB, H), lambda i: (i, 0)),
            pl.BlockSpec((H, H), lambda i: (0, 0)),
            pl.BlockSpec((H,), lambda i: (0,)),
        ],
        out_specs=[
            pl.BlockSpec((2, _NB, 128), lambda i: (0, i, 0)),
            pl.BlockSpec((2, H), lambda i: (0, 0)),
        ],
        out_shape=[
            jax.ShapeDtypeStruct((2, N, 128), _f32),
            jax.ShapeDtypeStruct((2, H), _f32),
        ],
    )(agg2, xin2, deg2, z, W, b)

    def bn_body(pre_ref, stat_ref, g_ref, be_ref, out_ref):
        mu = stat_ref[0:1] / N
        var = stat_ref[1:2] / N - mu * mu
        scale = lax.rsqrt(var + BN_EPS) * g_ref[...]
        shift = be_ref[...] - mu * scale
        pre = jnp.concatenate([pre_ref[0], pre_ref[1]], axis=1)
        hn = jnp.maximum(pre * scale + shift, 0.0)
        out_ref[0] = hn[:, :128]
        out_ref[1] = hn[:, 128:]

    return pl.pallas_call(
        bn_body,
        grid=(N // _NB,),
        in_specs=[
            pl.BlockSpec((2, _NB, 128), lambda i: (0, i, 0)),
            pl.BlockSpec((2, H), lambda i: (0, 0)),
            pl.BlockSpec((H,), lambda i: (0,)),
            pl.BlockSpec((H,), lambda i: (0,)),
        ],
        out_specs=pl.BlockSpec((2, _NB, 128), lambda i: (0, i, 0)),
        out_shape=jax.ShapeDtypeStruct((2, N, 128), _f32),
    )(pre2, stat, gamma, beta)


def _tc_gat_pre(h2, W, al, ar):
    K = h2.shape[0]

    def body(h_ref, W_ref, al_ref, ar_ref, feat_ref, elr_ref, emax_ref,
             mx_ref):
        i = pl.program_id(0)
        hcat = jnp.concatenate([h_ref[j] for j in range(K)], axis=1)
        for kk in range(8):
            feat_ref[kk] = jnp.dot(hcat, W_ref[:, kk * 128:(kk + 1) * 128],
                                   preferred_element_type=_f32)
        Wfull = W_ref[...]
        els, ers = [], []
        for hh in range(HEADS):
            Wh = Wfull[:, hh * H:(hh + 1) * H]
            wl = jnp.dot(Wh, al_ref[hh].reshape(H, 1),
                         preferred_element_type=_f32)
            wr = jnp.dot(Wh, ar_ref[hh].reshape(H, 1),
                         preferred_element_type=_f32)
            els.append(jnp.dot(hcat, wl, preferred_element_type=_f32))
            ers.append(jnp.dot(hcat, wr, preferred_element_type=_f32))
        el = jnp.concatenate(els, axis=1)
        er = jnp.concatenate(ers, axis=1)
        zpad = jnp.zeros((el.shape[0], 124), _f32)
        elr_ref[0] = jnp.concatenate([el, zpad], axis=1)
        elr_ref[1] = jnp.concatenate([er, zpad], axis=1)

        mel = jnp.max(el, axis=0)
        mer = jnp.max(er, axis=0)
        cur = jnp.concatenate([mel, mer]).reshape(1, 8)

        @pl.when(i == 0)
        def _():
            mx_ref[...] = jnp.full((1, 8), -jnp.inf, _f32)

        mx_ref[...] = jnp.maximum(mx_ref[...], cur)

        @pl.when(i == N // _NB - 1)
        def _():
            m = mx_ref[...]
            mh = jnp.maximum(m[:, :4] + m[:, 4:], 0.0)
            emax_ref[...] = jnp.concatenate(
                [mh, jnp.zeros((1, 12), _f32)], axis=1)

    return pl.pallas_call(
        body,
        grid=(N // _NB,),
        in_specs=[
            pl.BlockSpec((K, _NB, 128), lambda i: (0, i, 0)),
            pl.BlockSpec((K * 128, 8 * 128), lambda i: (0, 0)),
            pl.BlockSpec((HEADS, H), lambda i: (0, 0)),
            pl.BlockSpec((HEADS, H), lambda i: (0, 0)),
        ],
        out_specs=[
            pl.BlockSpec((8, _NB, 128), lambda i: (0, i, 0)),
            pl.BlockSpec((2, _NB, 128), lambda i: (0, i, 0)),
            pl.BlockSpec((1, 16), lambda i: (0, 0)),
        ],
        out_shape=[
            jax.ShapeDtypeStruct((8, N, 128), _f32),
            jax.ShapeDtypeStruct((2, N, 128), _f32),
            jax.ShapeDtypeStruct((1, 16), _f32),
        ],
        scratch_shapes=[pltpu.VMEM((1, 8), _f32)],
    )(h2, W, al, ar)


def _tc_gat_post(B8, spart2, bgat):
    def body(B_ref, sp_ref, b_ref, out_ref):
        s = sp_ref[0, :, :4] + sp_ref[1, :, :4]
        rinv = 1.0 / (s + 1e-16)
        for kk in range(8):
            hh = kk // 2
            out_ref[kk] = (B_ref[kk] * rinv[:, hh:hh + 1]
                           + b_ref[0, kk * 128:(kk + 1) * 128])

    return pl.pallas_call(
        body,
        grid=(N // _NB,),
        in_specs=[
            pl.BlockSpec((8, _NB, 128), lambda i: (0, i, 0)),
            pl.BlockSpec((2, _NB, 128), lambda i: (0, i, 0)),
            pl.BlockSpec((1, 8 * 128), lambda i: (0, 0)),
        ],
        out_specs=pl.BlockSpec((8, _NB, 128), lambda i: (0, i, 0)),
        out_shape=jax.ShapeDtypeStruct((8, N, 128), _f32),
    )(B8, spart2, bgat.reshape(1, 8 * 128))


def _tc_classifier(h8, W, b):
    def body(h_ref, W_ref, b_ref, out_ref):
        acc = b_ref[...] + jnp.zeros((h_ref.shape[1], CLASSES), _f32)
        for kk in range(8):
            acc = acc + jnp.dot(h_ref[kk], W_ref[kk * 128:(kk + 1) * 128],
                                preferred_element_type=_f32)
        out_ref[...] = acc

    return pl.pallas_call(
        body,
        grid=(N // _NB,),
        in_specs=[
            pl.BlockSpec((8, _NB, 128), lambda i: (0, i, 0)),
            pl.BlockSpec((8 * 128, CLASSES), lambda i: (0, 0)),
            pl.BlockSpec((CLASSES,), lambda i: (0,)),
        ],
        out_specs=pl.BlockSpec((_NB, CLASSES), lambda i: (i, 0)),
        out_shape=jax.ShapeDtypeStruct((N, CLASSES), _f32),
    )(h8, W, b)


def kernel(x, edge_index, params):
    p = params
    src = edge_index[0].astype(_i32)
    dst = edge_index[1].astype(_i32)

    z, x2 = _tc_vae(x, p['ae_W1'], p['ae_b1'], p['ae_Wmu'], p['ae_bmu'])

    deg2 = _sc_deg(dst).reshape(2, N, 128)

    agg0 = _sc_sage(x2.reshape(2 * N, 128), src, dst)
    h2 = _tc_sage_post(agg0.reshape(2, N, 128), x2, deg2, z,
                       p['sage_W0'], p['sage_b0'], p['bn_g0'], p['bn_b0'])
    agg1 = _sc_sage(h2.reshape(2 * N, 128), src, dst)
    h2 = _tc_sage_post(agg1.reshape(2, N, 128), h2, deg2, z,
                       p['sage_W1'], p['sage_b1'], p['bn_g1'], p['bn_b1'])

    for i in range(2):
        W = p['gat_W%d' % i]
        feat8, elr, emax = _tc_gat_pre(h2, W, p['gat_al%d' % i],
                                       p['gat_ar%d' % i])
        tnum, spart = _sc_gat_edge(elr.reshape(2 * N, 128), src, dst, emax)
        B8 = _sc_gat_msg(feat8.reshape(8 * N, 128), tnum, src, dst)
        h2 = _tc_gat_post(B8.reshape(8, N, 128), spart.reshape(2, N, 128),
                          p['gat_b%d' % i])

    return _tc_classifier(h2, p['cls_W'], p['cls_b'])
